# baseline (speedup 1.0000x reference)
import dataclasses

import jax
import jax.numpy as jnp
from jax import lax
from jax.experimental import pallas as pl
from jax.experimental.pallas import tpu as pltpu
from jax.experimental.pallas import tpu_sc as plsc

T = 256
S = 16
HIDDEN = 1024
BATCH = 4

NC = 2
NS = 16
LANES = 16
NW = NC * NS
TPW = T // NW
NCHUNK = HIDDEN // LANES


def _stats_sc_body(t_hbm, s_hbm, mean_hbm, var_hbm, t_v, s_v, m_v, v_v,
                   sem_t, sem_s):
    cid = lax.axis_index("c")
    sid = lax.axis_index("s")
    wid = sid * NC + cid
    base_t = wid * TPW
    cp_t = pltpu.async_copy(t_hbm.at[pl.ds(base_t, TPW)], t_v, sem_t)
    cp_s = pltpu.async_copy(s_hbm, s_v, sem_s)
    cp_t.wait()
    cp_s.wait()

    zero16 = jnp.zeros((LANES,), jnp.float32)
    inv_h = jnp.float32(1.0 / HIDDEN)

    def s_body(c, carry):
        new = []
        for s in range(S):
            sv = s_v[s, pl.ds(c * LANES, LANES)]
            new.append(carry[2 * s] + sv)
            new.append(carry[2 * s + 1] + sv * sv)
        return tuple(new)

    s_acc = lax.fori_loop(0, NCHUNK, s_body, (zero16,) * (2 * S))
    ss = [jnp.sum(s_acc[2 * s]) for s in range(S)]
    ss2 = [jnp.sum(s_acc[2 * s + 1]) for s in range(S)]

    lane = lax.iota(jnp.int32, LANES)
    for t0 in range(0, TPW, 2):
        def t_body(c, carry):
            new = list(carry)
            for u in range(2):
                cc = 2 * c + u
                tva = t_v[t0, pl.ds(cc * LANES, LANES)]
                tvb = t_v[t0 + 1, pl.ds(cc * LANES, LANES)]
                new[0] = new[0] + tva
                new[1] = new[1] + tva * tva
                new[2] = new[2] + tvb
                new[3] = new[3] + tvb * tvb
                for s in range(S):
                    sv = s_v[s, pl.ds(cc * LANES, LANES)]
                    new[4 + 2 * s] = new[4 + 2 * s] + tva * sv
                    new[4 + 2 * s + 1] = new[4 + 2 * s + 1] + tvb * sv
            return tuple(new)

        acc = lax.fori_loop(0, NCHUNK // 2, t_body, (zero16,) * (4 + 2 * S))
        for k in range(2):
            t = t0 + k
            st = jnp.sum(acc[2 * k])
            st2 = jnp.sum(acc[2 * k + 1])
            mean_row = zero16
            var_row = zero16
            for s in range(S):
                cross = jnp.sum(acc[4 + 2 * s + k])
                mean = (st + ss[s]) * inv_h
                e2 = (st2 + ss2[s] + 2.0 * cross) * inv_h
                var = e2 - mean * mean
                msk = lane == s
                mean_row = jnp.where(msk, mean, mean_row)
                var_row = jnp.where(msk, var, var_row)
            m_v[t] = mean_row
            v_v[t] = var_row

    cp_m = pltpu.async_copy(m_v, mean_hbm.at[pl.ds(base_t, TPW)], sem_t)
    cp_v = pltpu.async_copy(v_v, var_hbm.at[pl.ds(base_t, TPW)], sem_s)
    cp_m.wait()
    cp_v.wait()


def _stats_sparsecore(t_emb, s_emb):
    cp = pltpu.CompilerParams()
    if "needs_layout_passes" in pltpu.CompilerParams.__dataclass_fields__:
        cp = dataclasses.replace(cp, needs_layout_passes=False)
    kern = pl.kernel(
        _stats_sc_body,
        compiler_params=cp,
        out_type=(
            jax.ShapeDtypeStruct((T, S), jnp.float32),
            jax.ShapeDtypeStruct((T, S), jnp.float32),
        ),
        mesh=plsc.VectorSubcoreMesh(core_axis_name="c", subcore_axis_name="s"),
        scratch_types=[
            pltpu.VMEM((TPW, HIDDEN), jnp.float32),
            pltpu.VMEM((S, HIDDEN), jnp.float32),
            pltpu.VMEM((TPW, S), jnp.float32),
            pltpu.VMEM((TPW, S), jnp.float32),
            pltpu.SemaphoreType.DMA,
            pltpu.SemaphoreType.DMA,
        ],
    )
    return kern(t_emb, s_emb)


T_BLK = 32


def _fused_tc_body(in_ref, t_ref, s_ref, mean_ref, var_ref,
                   scale_ref, bias_ref, out_ref):
    pos = t_ref[...][:, None, :] + s_ref[...][None, :, :]
    mean = mean_ref[...][:, :, None]
    rstd = lax.rsqrt(var_ref[...][:, :, None] + 1e-6)
    y = (pos - mean) * rstd * scale_ref[...] + bias_ref[...]
    out_ref[...] = in_ref[...] + y[None, None]


def _fused_tc(inputs, t_emb, s_emb, mean, var, ln_scale, ln_bias):
    in5 = inputs.reshape(BATCH, 1, T, S, HIDDEN)
    grid = (T // T_BLK,)
    out5 = pl.pallas_call(
        _fused_tc_body,
        grid=grid,
        in_specs=[
            pl.BlockSpec((BATCH, 1, T_BLK, S, HIDDEN), lambda i: (0, 0, i, 0, 0)),
            pl.BlockSpec((T_BLK, HIDDEN), lambda i: (i, 0)),
            pl.BlockSpec((S, HIDDEN), lambda i: (0, 0)),
            pl.BlockSpec((T_BLK, S), lambda i: (i, 0)),
            pl.BlockSpec((T_BLK, S), lambda i: (i, 0)),
            pl.BlockSpec((1, HIDDEN), lambda i: (0, 0)),
            pl.BlockSpec((1, HIDDEN), lambda i: (0, 0)),
        ],
        out_specs=pl.BlockSpec((BATCH, 1, T_BLK, S, HIDDEN),
                               lambda i: (0, 0, i, 0, 0)),
        out_shape=jax.ShapeDtypeStruct((BATCH, 1, T, S, HIDDEN), jnp.float32),
    )(in5, t_emb, s_emb, mean, var,
      ln_scale.reshape(1, HIDDEN), ln_bias.reshape(1, HIDDEN))
    return out5.reshape(BATCH, 1, T * S, HIDDEN)


def kernel(inputs, temporal_embedding, spectoral_embedding, ln_scale, ln_bias):
    mean, var = _stats_sparsecore(temporal_embedding, spectoral_embedding)
    return _fused_tc(inputs, temporal_embedding, spectoral_embedding,
                     mean, var, ln_scale, ln_bias)


# speedup vs baseline: 1.0373x; 1.0373x over previous
import dataclasses

import jax
import jax.numpy as jnp
from jax import lax
from jax.experimental import pallas as pl
from jax.experimental.pallas import tpu as pltpu
from jax.experimental.pallas import tpu_sc as plsc

T = 256
S = 16
HIDDEN = 1024
BATCH = 4

NC = 2
NS = 16
LANES = 16
NW = NC * NS
TPW = T // NW
NCHUNK = HIDDEN // LANES


def _stats_sc_body(t_hbm, s_hbm, mean_hbm, var_hbm, t_v, s_v, m_v, v_v,
                   sem_t, sem_s):
    cid = lax.axis_index("c")
    sid = lax.axis_index("s")
    wid = sid * NC + cid
    base_t = wid * TPW
    cp_t = pltpu.async_copy(t_hbm.at[pl.ds(base_t, TPW)], t_v, sem_t)
    cp_s = pltpu.async_copy(s_hbm, s_v, sem_s)
    cp_t.wait()
    cp_s.wait()

    zero16 = jnp.zeros((LANES,), jnp.float32)
    inv_h = jnp.float32(1.0 / HIDDEN)

    def s_body(c, carry):
        new = []
        for s in range(S):
            sv = s_v[s, pl.ds(c * LANES, LANES)]
            new.append(carry[2 * s] + sv)
            new.append(carry[2 * s + 1] + sv * sv)
        return tuple(new)

    s_acc = lax.fori_loop(0, NCHUNK, s_body, (zero16,) * (2 * S))
    ss = [jnp.sum(s_acc[2 * s]) for s in range(S)]
    ss2 = [jnp.sum(s_acc[2 * s + 1]) for s in range(S)]

    lane = lax.iota(jnp.int32, LANES)
    for t0 in range(0, TPW, 2):
        def t_body(c, carry):
            tva = t_v[t0, pl.ds(c * LANES, LANES)]
            tvb = t_v[t0 + 1, pl.ds(c * LANES, LANES)]
            new = [carry[0] + tva, carry[1] + tva * tva,
                   carry[2] + tvb, carry[3] + tvb * tvb]
            for s in range(S):
                sv = s_v[s, pl.ds(c * LANES, LANES)]
                new.append(carry[4 + 2 * s] + tva * sv)
                new.append(carry[4 + 2 * s + 1] + tvb * sv)
            return tuple(new)

        acc = lax.fori_loop(0, NCHUNK, t_body, (zero16,) * (4 + 2 * S))
        for k in range(2):
            t = t0 + k
            st = jnp.sum(acc[2 * k])
            st2 = jnp.sum(acc[2 * k + 1])
            mean_row = zero16
            var_row = zero16
            for s in range(S):
                cross = jnp.sum(acc[4 + 2 * s + k])
                mean = (st + ss[s]) * inv_h
                e2 = (st2 + ss2[s] + 2.0 * cross) * inv_h
                var = e2 - mean * mean
                msk = lane == s
                mean_row = jnp.where(msk, mean, mean_row)
                var_row = jnp.where(msk, var, var_row)
            m_v[t] = mean_row
            v_v[t] = var_row

    pltpu.sync_copy(m_v, mean_hbm.at[pl.ds(base_t, TPW)])
    pltpu.sync_copy(v_v, var_hbm.at[pl.ds(base_t, TPW)])


def _stats_sparsecore(t_emb, s_emb):
    cp = pltpu.CompilerParams()
    if "needs_layout_passes" in pltpu.CompilerParams.__dataclass_fields__:
        cp = dataclasses.replace(cp, needs_layout_passes=False)
    kern = pl.kernel(
        _stats_sc_body,
        compiler_params=cp,
        out_type=(
            jax.ShapeDtypeStruct((T, S), jnp.float32),
            jax.ShapeDtypeStruct((T, S), jnp.float32),
        ),
        mesh=plsc.VectorSubcoreMesh(core_axis_name="c", subcore_axis_name="s"),
        scratch_types=[
            pltpu.VMEM((TPW, HIDDEN), jnp.float32),
            pltpu.VMEM((S, HIDDEN), jnp.float32),
            pltpu.VMEM((TPW, S), jnp.float32),
            pltpu.VMEM((TPW, S), jnp.float32),
            pltpu.SemaphoreType.DMA,
            pltpu.SemaphoreType.DMA,
        ],
    )
    return kern(t_emb, s_emb)


T_BLK = 32


def _fused_tc_body(in_ref, t_ref, s_ref, mean_ref, var_ref,
                   scale_ref, bias_ref, out_ref):
    pos = t_ref[...][:, None, :] + s_ref[...][None, :, :]
    mean = mean_ref[...][:, :, None]
    rstd = lax.rsqrt(var_ref[...][:, :, None] + 1e-6)
    y = (pos - mean) * rstd * scale_ref[...] + bias_ref[...]
    out_ref[...] = in_ref[...] + y[None, None]


def _fused_tc(inputs, t_emb, s_emb, mean, var, ln_scale, ln_bias):
    in5 = inputs.reshape(BATCH, 1, T, S, HIDDEN)
    grid = (T // T_BLK,)
    out5 = pl.pallas_call(
        _fused_tc_body,
        grid=grid,
        in_specs=[
            pl.BlockSpec((BATCH, 1, T_BLK, S, HIDDEN), lambda i: (0, 0, i, 0, 0)),
            pl.BlockSpec((T_BLK, HIDDEN), lambda i: (i, 0)),
            pl.BlockSpec((S, HIDDEN), lambda i: (0, 0)),
            pl.BlockSpec((T_BLK, S), lambda i: (i, 0)),
            pl.BlockSpec((T_BLK, S), lambda i: (i, 0)),
            pl.BlockSpec((1, HIDDEN), lambda i: (0, 0)),
            pl.BlockSpec((1, HIDDEN), lambda i: (0, 0)),
        ],
        out_specs=pl.BlockSpec((BATCH, 1, T_BLK, S, HIDDEN),
                               lambda i: (0, 0, i, 0, 0)),
        out_shape=jax.ShapeDtypeStruct((BATCH, 1, T, S, HIDDEN), jnp.float32),
    )(in5, t_emb, s_emb, mean, var,
      ln_scale.reshape(1, HIDDEN), ln_bias.reshape(1, HIDDEN))
    return out5.reshape(BATCH, 1, T * S, HIDDEN)


def kernel(inputs, temporal_embedding, spectoral_embedding, ln_scale, ln_bias):
    mean, var = _stats_sparsecore(temporal_embedding, spectoral_embedding)
    return _fused_tc(inputs, temporal_embedding, spectoral_embedding,
                     mean, var, ln_scale, ln_bias)
